# initial kernel scaffold (unmeasured)
import jax
import jax.numpy as jnp
from jax import lax
from jax.experimental import pallas as pl
from jax.experimental.pallas import tpu as pltpu

EPS = 1e-5
ROW_BLOCK = 1024


def _partial_body(x_ref, dy_ref, out_ref):
    i = pl.program_id(0)
    x = x_ref[:, :]
    dy = dy_ref[:, :]
    mu = jnp.mean(x, axis=1, keepdims=True)
    xc = x - mu
    var = jnp.mean(xc * xc, axis=1, keepdims=True)
    xhat = xc * lax.rsqrt(var + EPS)
    dgamma = jnp.sum(dy * xhat, axis=0)[None, :]
    dbeta = jnp.sum(dy, axis=0)[None, :]
    partial = jnp.concatenate([dgamma, dbeta], axis=0)

    @pl.when(i == 0)
    def _():
        out_ref[:, :] = partial

    @pl.when(i != 0)
    def _():
        out_ref[:, :] = out_ref[:, :] + partial


def _allreduce_x_body(p_ref, out_ref, comm_ref, send_sem, recv_sem):
    my_x = lax.axis_index("x")
    my_y = lax.axis_index("y")
    my_z = lax.axis_index("z")
    partner = (1 - my_x, my_y, my_z)

    barrier = pltpu.get_barrier_semaphore()
    pl.semaphore_signal(
        barrier, inc=1, device_id=partner, device_id_type=pl.DeviceIdType.MESH
    )
    pl.semaphore_wait(barrier, 1)

    rdma = pltpu.make_async_remote_copy(
        src_ref=p_ref,
        dst_ref=comm_ref,
        send_sem=send_sem,
        recv_sem=recv_sem,
        device_id=partner,
        device_id_type=pl.DeviceIdType.MESH,
    )
    rdma.start()
    rdma.wait()
    out_ref[:, :] = p_ref[:, :] + comm_ref[:, :]


def kernel(x, dy, gamma):
    del gamma
    m, d = x.shape
    n_blocks = m // ROW_BLOCK

    partial = pl.pallas_call(
        _partial_body,
        grid=(n_blocks,),
        in_specs=[
            pl.BlockSpec((ROW_BLOCK, d), lambda i: (i, 0)),
            pl.BlockSpec((ROW_BLOCK, d), lambda i: (i, 0)),
        ],
        out_specs=pl.BlockSpec((2, d), lambda i: (0, 0)),
        out_shape=jax.ShapeDtypeStruct((2, d), jnp.float32),
    )(x, dy)

    return pl.pallas_call(
        _allreduce_x_body,
        out_shape=jax.ShapeDtypeStruct((2, d), jnp.float32),
        in_specs=[pl.BlockSpec(memory_space=pltpu.VMEM)],
        out_specs=pl.BlockSpec(memory_space=pltpu.VMEM),
        scratch_shapes=[
            pltpu.VMEM((2, d), jnp.float32),
            pltpu.SemaphoreType.DMA,
            pltpu.SemaphoreType.DMA,
        ],
        compiler_params=pltpu.CompilerParams(collective_id=0),
    )(partial)


# baseline (device time: 26420 ns/iter reference)
import jax
import jax.numpy as jnp
from jax import lax
from jax.experimental import pallas as pl
from jax.experimental.pallas import tpu as pltpu

EPS = 1e-5
ROW_BLOCK = 512


def _partial_body(x_ref, dy_ref, out_ref):
    i = pl.program_id(0)
    x = x_ref[:, :]
    dy = dy_ref[:, :]
    mu = jnp.mean(x, axis=1, keepdims=True)
    xc = x - mu
    var = jnp.mean(xc * xc, axis=1, keepdims=True)
    xhat = xc * lax.rsqrt(var + EPS)
    dgamma = jnp.sum(dy * xhat, axis=0)[None, :]
    dbeta = jnp.sum(dy, axis=0)[None, :]
    partial = jnp.concatenate([dgamma, dbeta], axis=0)

    @pl.when(i == 0)
    def _():
        out_ref[:, :] = partial

    @pl.when(i != 0)
    def _():
        out_ref[:, :] = out_ref[:, :] + partial


def _allreduce_x_body(p_ref, out_ref, comm_ref, send_sem, recv_sem):
    my_x = lax.axis_index("x")
    my_y = lax.axis_index("y")
    my_z = lax.axis_index("z")
    partner = (1 - my_x, my_y, my_z)

    barrier = pltpu.get_barrier_semaphore()
    pl.semaphore_signal(
        barrier, inc=1, device_id=partner, device_id_type=pl.DeviceIdType.MESH
    )
    pl.semaphore_wait(barrier, 1)

    rdma = pltpu.make_async_remote_copy(
        src_ref=p_ref,
        dst_ref=comm_ref,
        send_sem=send_sem,
        recv_sem=recv_sem,
        device_id=partner,
        device_id_type=pl.DeviceIdType.MESH,
    )
    rdma.start()
    rdma.wait()
    out_ref[:, :] = p_ref[:, :] + comm_ref[:, :]


def kernel(x, dy, gamma):
    del gamma
    m, d = x.shape
    n_blocks = m // ROW_BLOCK

    partial = pl.pallas_call(
        _partial_body,
        grid=(n_blocks,),
        in_specs=[
            pl.BlockSpec((ROW_BLOCK, d), lambda i: (i, 0)),
            pl.BlockSpec((ROW_BLOCK, d), lambda i: (i, 0)),
        ],
        out_specs=pl.BlockSpec((2, d), lambda i: (0, 0)),
        out_shape=jax.ShapeDtypeStruct((2, d), jnp.float32),
    )(x, dy)

    return pl.pallas_call(
        _allreduce_x_body,
        out_shape=jax.ShapeDtypeStruct((2, d), jnp.float32),
        in_specs=[pl.BlockSpec(memory_space=pltpu.VMEM)],
        out_specs=pl.BlockSpec(memory_space=pltpu.VMEM),
        scratch_shapes=[
            pltpu.VMEM((2, d), jnp.float32),
            pltpu.SemaphoreType.DMA,
            pltpu.SemaphoreType.DMA,
        ],
        compiler_params=pltpu.CompilerParams(collective_id=0),
    )(partial)


# device time: 15221 ns/iter; 1.7358x vs baseline; 1.7358x over previous
import jax
import jax.numpy as jnp
from jax import lax
from jax.experimental import pallas as pl
from jax.experimental.pallas import tpu as pltpu

EPS = 1e-5
YDIM = 4
ZDIM = 4
N_PLANE = YDIM * ZDIM
PLANE = [(yj, zj) for yj in range(YDIM) for zj in range(ZDIM)]


def _body(
    x_hbm,
    dy_hbm,
    out_ref,
    xb,
    dyb,
    sendbuf1,
    comm1,
    sendbuf2,
    comm2,
    local_sems,
    sem1_send,
    sem1_recv,
    send_sems,
    recv_sems,
):
    my_x = lax.axis_index("x")
    my_y = lax.axis_index("y")
    my_z = lax.axis_index("z")
    k = my_y * ZDIM + my_z
    rows = xb.shape[0]

    barrier = pltpu.get_barrier_semaphore()
    pl.semaphore_signal(
        barrier,
        inc=1,
        device_id=(1 - my_x, my_y, my_z),
        device_id_type=pl.DeviceIdType.MESH,
    )
    for j, (yj, zj) in enumerate(PLANE):

        @pl.when(j != k)
        def _(yj=yj, zj=zj):
            pl.semaphore_signal(
                barrier,
                inc=1,
                device_id=(my_x, yj, zj),
                device_id_type=pl.DeviceIdType.MESH,
            )

    cp_x = pltpu.make_async_copy(
        x_hbm.at[pl.ds(k * rows, rows), :], xb, local_sems.at[0]
    )
    cp_dy = pltpu.make_async_copy(
        dy_hbm.at[pl.ds(k * rows, rows), :], dyb, local_sems.at[1]
    )
    cp_x.start()
    cp_dy.start()
    cp_x.wait()
    cp_dy.wait()

    x = xb[:, :]
    dy = dyb[:, :]
    mu = jnp.mean(x, axis=1, keepdims=True)
    xc = x - mu
    var = jnp.mean(xc * xc, axis=1, keepdims=True)
    xhat = xc * lax.rsqrt(var + EPS)
    dgamma = jnp.sum(dy * xhat, axis=0)[None, :]
    dbeta = jnp.sum(dy, axis=0)[None, :]
    sendbuf1[:, :] = jnp.concatenate([dgamma, dbeta], axis=0)

    pl.semaphore_wait(barrier, N_PLANE)

    rdma1 = pltpu.make_async_remote_copy(
        src_ref=sendbuf1,
        dst_ref=comm1,
        send_sem=sem1_send,
        recv_sem=sem1_recv,
        device_id=(1 - my_x, my_y, my_z),
        device_id_type=pl.DeviceIdType.MESH,
    )
    rdma1.start()
    rdma1.wait()
    xsum = sendbuf1[:, :] + comm1[:, :]
    sendbuf2[:, :] = xsum

    for j, (yj, zj) in enumerate(PLANE):

        @pl.when(j != k)
        def _(j=j, yj=yj, zj=zj):
            rdma = pltpu.make_async_remote_copy(
                src_ref=sendbuf2,
                dst_ref=comm2.at[k],
                send_sem=send_sems.at[j],
                recv_sem=recv_sems.at[k],
                device_id=(my_x, yj, zj),
                device_id_type=pl.DeviceIdType.MESH,
            )
            rdma.start()

    comm2[pl.ds(k, 1), :, :] = xsum[None, :, :]

    for j in range(N_PLANE):

        @pl.when(j != k)
        def _(j=j):
            desc = pltpu.make_async_remote_copy(
                src_ref=sendbuf2,
                dst_ref=comm2.at[j],
                send_sem=send_sems.at[j],
                recv_sem=recv_sems.at[j],
                device_id=(my_x, my_y, my_z),
                device_id_type=pl.DeviceIdType.MESH,
            )
            desc.wait_send()
            desc.wait_recv()

    out_ref[:, :] = jnp.sum(comm2[:, :, :], axis=0)


def kernel(x, dy, gamma):
    del gamma
    m, d = x.shape
    rows = m // N_PLANE

    return pl.pallas_call(
        _body,
        out_shape=jax.ShapeDtypeStruct((2, d), jnp.float32),
        in_specs=[
            pl.BlockSpec(memory_space=pl.ANY),
            pl.BlockSpec(memory_space=pl.ANY),
        ],
        out_specs=pl.BlockSpec(memory_space=pltpu.VMEM),
        scratch_shapes=[
            pltpu.VMEM((rows, d), jnp.float32),
            pltpu.VMEM((rows, d), jnp.float32),
            pltpu.VMEM((2, d), jnp.float32),
            pltpu.VMEM((2, d), jnp.float32),
            pltpu.VMEM((2, d), jnp.float32),
            pltpu.VMEM((N_PLANE, 2, d), jnp.float32),
            pltpu.SemaphoreType.DMA((2,)),
            pltpu.SemaphoreType.DMA,
            pltpu.SemaphoreType.DMA,
            pltpu.SemaphoreType.DMA((N_PLANE,)),
            pltpu.SemaphoreType.DMA((N_PLANE,)),
        ],
        compiler_params=pltpu.CompilerParams(collective_id=0),
    )(x, dy)


# device time: 14913 ns/iter; 1.7716x vs baseline; 1.0207x over previous
import functools

import jax
import jax.numpy as jnp
from jax import lax
from jax.experimental import pallas as pl
from jax.experimental.pallas import tpu as pltpu

EPS = 1e-5
YDIM = 4
ZDIM = 4
N_PLANE = YDIM * ZDIM
PLANE = [(yj, zj) for yj in range(YDIM) for zj in range(ZDIM)]


def _body(
    x_hbm,
    dy_hbm,
    out_ref,
    xb,
    dyb,
    sendbuf1,
    comm1,
    sendbuf2,
    comm2,
    local_sems,
    sem1_send,
    sem1_recv,
    send_sems,
    recv_sems,
):
    my_x = lax.axis_index("x")
    my_y = lax.axis_index("y")
    my_z = lax.axis_index("z")
    k = my_y * ZDIM + my_z
    rows = xb.shape[0]

    xbar = pltpu.get_barrier_semaphore()

    @functools.partial(
        pl.run_scoped, plane_bar=pltpu.SemaphoreType.REGULAR
    )
    def _(plane_bar):
        pl.semaphore_signal(
            xbar,
            inc=1,
            device_id=(1 - my_x, my_y, my_z),
            device_id_type=pl.DeviceIdType.MESH,
        )
        for j, (yj, zj) in enumerate(PLANE):

            @pl.when(j != k)
            def _(yj=yj, zj=zj):
                pl.semaphore_signal(
                    plane_bar,
                    inc=1,
                    device_id=(my_x, yj, zj),
                    device_id_type=pl.DeviceIdType.MESH,
                )

        cp_x = pltpu.make_async_copy(
            x_hbm.at[pl.ds(k * rows, rows), :], xb, local_sems.at[0]
        )
        cp_dy = pltpu.make_async_copy(
            dy_hbm.at[pl.ds(k * rows, rows), :], dyb, local_sems.at[1]
        )
        cp_x.start()
        cp_dy.start()
        cp_x.wait()
        cp_dy.wait()

        x = xb[:, :]
        dy = dyb[:, :]
        mu = jnp.mean(x, axis=1, keepdims=True)
        xc = x - mu
        var = jnp.mean(xc * xc, axis=1, keepdims=True)
        xhat = xc * lax.rsqrt(var + EPS)
        dgamma = jnp.sum(dy * xhat, axis=0)[None, :]
        dbeta = jnp.sum(dy, axis=0)[None, :]
        sendbuf1[:, :] = jnp.concatenate([dgamma, dbeta], axis=0).astype(
            jnp.bfloat16
        )

        pl.semaphore_wait(xbar, 1)
        rdma1 = pltpu.make_async_remote_copy(
            src_ref=sendbuf1,
            dst_ref=comm1,
            send_sem=sem1_send,
            recv_sem=sem1_recv,
            device_id=(1 - my_x, my_y, my_z),
            device_id_type=pl.DeviceIdType.MESH,
        )
        rdma1.start()
        rdma1.wait()
        xsum = sendbuf1[:, :].astype(jnp.float32) + comm1[:, :].astype(
            jnp.float32
        )
        sendbuf2[:, :] = xsum.astype(jnp.bfloat16)

        pl.semaphore_wait(plane_bar, N_PLANE - 1)
        for j, (yj, zj) in enumerate(PLANE):

            @pl.when(j != k)
            def _(j=j, yj=yj, zj=zj):
                rdma = pltpu.make_async_remote_copy(
                    src_ref=sendbuf2,
                    dst_ref=comm2.at[k],
                    send_sem=send_sems.at[j],
                    recv_sem=recv_sems.at[k],
                    device_id=(my_x, yj, zj),
                    device_id_type=pl.DeviceIdType.MESH,
                )
                rdma.start()

        comm2[pl.ds(k, 1), :, :] = xsum.astype(jnp.bfloat16)[None, :, :]

        for j in range(N_PLANE):

            @pl.when(j != k)
            def _(j=j):
                desc = pltpu.make_async_remote_copy(
                    src_ref=sendbuf2,
                    dst_ref=comm2.at[j],
                    send_sem=send_sems.at[j],
                    recv_sem=recv_sems.at[j],
                    device_id=(my_x, my_y, my_z),
                    device_id_type=pl.DeviceIdType.MESH,
                )
                desc.wait_send()
                desc.wait_recv()

        out_ref[:, :] = jnp.sum(
            comm2[:, :, :].astype(jnp.float32), axis=0
        )


def kernel(x, dy, gamma):
    del gamma
    m, d = x.shape
    rows = m // N_PLANE

    return pl.pallas_call(
        _body,
        out_shape=jax.ShapeDtypeStruct((2, d), jnp.float32),
        in_specs=[
            pl.BlockSpec(memory_space=pl.ANY),
            pl.BlockSpec(memory_space=pl.ANY),
        ],
        out_specs=pl.BlockSpec(memory_space=pltpu.VMEM),
        scratch_shapes=[
            pltpu.VMEM((rows, d), jnp.float32),
            pltpu.VMEM((rows, d), jnp.float32),
            pltpu.VMEM((2, d), jnp.bfloat16),
            pltpu.VMEM((2, d), jnp.bfloat16),
            pltpu.VMEM((2, d), jnp.bfloat16),
            pltpu.VMEM((N_PLANE, 2, d), jnp.bfloat16),
            pltpu.SemaphoreType.DMA((2,)),
            pltpu.SemaphoreType.DMA,
            pltpu.SemaphoreType.DMA,
            pltpu.SemaphoreType.DMA((N_PLANE,)),
            pltpu.SemaphoreType.DMA((N_PLANE,)),
        ],
        compiler_params=pltpu.CompilerParams(collective_id=0),
    )(x, dy)


# device time: 14909 ns/iter; 1.7721x vs baseline; 1.0003x over previous
import functools

import jax
import jax.numpy as jnp
from jax import lax
from jax.experimental import pallas as pl
from jax.experimental.pallas import tpu as pltpu

EPS = 1e-5
YDIM = 4
ZDIM = 4
N_PLANE = YDIM * ZDIM
PLANE = [(yj, zj) for yj in range(YDIM) for zj in range(ZDIM)]


def _body(
    x_hbm,
    dy_hbm,
    out_ref,
    xb,
    dyb,
    sendbuf1,
    comm1,
    sendbuf2,
    comm2,
    local_sems,
    sem1_send,
    sem1_recv,
    send_sems,
    recv_sems,
):
    my_x = lax.axis_index("x")
    my_y = lax.axis_index("y")
    my_z = lax.axis_index("z")
    k = my_y * ZDIM + my_z
    rows = xb.shape[0]

    xbar = pltpu.get_barrier_semaphore()

    @functools.partial(
        pl.run_scoped, plane_bar=pltpu.SemaphoreType.REGULAR
    )
    def _(plane_bar):
        pl.semaphore_signal(
            xbar,
            inc=1,
            device_id=(1 - my_x, my_y, my_z),
            device_id_type=pl.DeviceIdType.MESH,
        )
        for j, (yj, zj) in enumerate(PLANE):

            @pl.when(j != k)
            def _(yj=yj, zj=zj):
                pl.semaphore_signal(
                    plane_bar,
                    inc=1,
                    device_id=(my_x, yj, zj),
                    device_id_type=pl.DeviceIdType.MESH,
                )

        with jax.named_scope("dma_compute"):
            cp_x = pltpu.make_async_copy(
                x_hbm.at[pl.ds(k * rows, rows), :], xb, local_sems.at[0]
            )
            cp_dy = pltpu.make_async_copy(
                dy_hbm.at[pl.ds(k * rows, rows), :], dyb, local_sems.at[1]
            )
            cp_x.start()
            cp_dy.start()
            cp_x.wait()
            cp_dy.wait()

            x = xb[:, :]
            dy = dyb[:, :]
            mu = jnp.mean(x, axis=1, keepdims=True)
            xc = x - mu
            var = jnp.mean(xc * xc, axis=1, keepdims=True)
            xhat = xc * lax.rsqrt(var + EPS)
            dgamma = jnp.sum(dy * xhat, axis=0)[None, :]
            dbeta = jnp.sum(dy, axis=0)[None, :]
            sendbuf1[:, :] = jnp.concatenate([dgamma, dbeta], axis=0).astype(
                jnp.bfloat16
            )

        with jax.named_scope("xbar_wait"):
            pl.semaphore_wait(xbar, 1)
        with jax.named_scope("stage1"):
            rdma1 = pltpu.make_async_remote_copy(
                src_ref=sendbuf1,
                dst_ref=comm1,
                send_sem=sem1_send,
                recv_sem=sem1_recv,
                device_id=(1 - my_x, my_y, my_z),
                device_id_type=pl.DeviceIdType.MESH,
            )
            rdma1.start()
            rdma1.wait()
            xsum = sendbuf1[:, :].astype(jnp.float32) + comm1[:, :].astype(
                jnp.float32
            )
            sendbuf2[:, :] = xsum.astype(jnp.bfloat16)

        with jax.named_scope("planebar_wait"):
            pl.semaphore_wait(plane_bar, N_PLANE - 1)
        with jax.named_scope("stage2_send"):
            for j, (yj, zj) in enumerate(PLANE):

                @pl.when(j != k)
                def _(j=j, yj=yj, zj=zj):
                    rdma = pltpu.make_async_remote_copy(
                        src_ref=sendbuf2,
                        dst_ref=comm2.at[k],
                        send_sem=send_sems.at[j],
                        recv_sem=recv_sems.at[k],
                        device_id=(my_x, yj, zj),
                        device_id_type=pl.DeviceIdType.MESH,
                    )
                    rdma.start()

            comm2[pl.ds(k, 1), :, :] = xsum.astype(jnp.bfloat16)[None, :, :]

        with jax.named_scope("stage2_wait"):
            for j in range(N_PLANE):

                @pl.when(j != k)
                def _(j=j):
                    desc = pltpu.make_async_remote_copy(
                        src_ref=sendbuf2,
                        dst_ref=comm2.at[j],
                        send_sem=send_sems.at[j],
                        recv_sem=recv_sems.at[j],
                        device_id=(my_x, my_y, my_z),
                        device_id_type=pl.DeviceIdType.MESH,
                    )
                    desc.wait_send()
                    desc.wait_recv()

        with jax.named_scope("final_sum"):
            out_ref[:, :] = jnp.sum(
                comm2[:, :, :].astype(jnp.float32), axis=0
            )


def kernel(x, dy, gamma):
    del gamma
    m, d = x.shape
    rows = m // N_PLANE

    return pl.pallas_call(
        _body,
        out_shape=jax.ShapeDtypeStruct((2, d), jnp.float32),
        in_specs=[
            pl.BlockSpec(memory_space=pl.ANY),
            pl.BlockSpec(memory_space=pl.ANY),
        ],
        out_specs=pl.BlockSpec(memory_space=pltpu.VMEM),
        scratch_shapes=[
            pltpu.VMEM((rows, d), jnp.float32),
            pltpu.VMEM((rows, d), jnp.float32),
            pltpu.VMEM((2, d), jnp.bfloat16),
            pltpu.VMEM((2, d), jnp.bfloat16),
            pltpu.VMEM((2, d), jnp.bfloat16),
            pltpu.VMEM((N_PLANE, 2, d), jnp.bfloat16),
            pltpu.SemaphoreType.DMA((2,)),
            pltpu.SemaphoreType.DMA,
            pltpu.SemaphoreType.DMA,
            pltpu.SemaphoreType.DMA((N_PLANE,)),
            pltpu.SemaphoreType.DMA((N_PLANE,)),
        ],
        compiler_params=pltpu.CompilerParams(collective_id=0),
    )(x, dy)
